# baseline (speedup 1.0000x reference)
import functools

import jax
import jax.numpy as jnp
import numpy as np
from jax import lax
from jax.experimental import pallas as pl
from jax.experimental.pallas import tpu as pltpu
from jax.experimental.pallas import tpu_sc as plsc

_CARDS = [100000] * 26
_F = len(_CARDS)
_D = 32
_B = 16384
_R = sum(_CARDS)
_RP = _R // 4

_NC, _NS, _L = 2, 16, 16
_NW = _NC * _NS

_PAIRS = _F * (_B // 128)
_PER_W = _PAIRS // _NW


def _body(x_hbm, bias_hbm, emb_hbm, out_hbm,
          x_v, idx_v, wide_v, tiles_v, bias_v, sem):
    cid = lax.axis_index("c")
    sid = lax.axis_index("s")
    wid = sid * _NC + cid

    out2 = out_hbm.reshape(_F * _D, _B)

    pltpu.sync_copy(bias_hbm, bias_v)
    lanes = lax.iota(jnp.int32, _L)

    def pair(t, carry):
        p = wid * _PER_W + t
        f = p // 128
        tc = p % 128
        pltpu.sync_copy(x_hbm.at[pl.ds(f * _B + tc * 128, 128)], x_v)
        slot = []
        for k in range(8):
            r = (x_v[pl.ds(k * _L, _L)] + (f * 100000).astype(jnp.float32)
                 ).astype(jnp.int32)
            idx_v[0, pl.ds(k * _L, _L)] = r >> 2
            slot.append((r & 3) << 5)
        cop = pltpu.async_copy(emb_hbm.at[idx_v.at[0]], wide_v, sem)
        cop.wait()
        def pick(j, c2):
            bvec = bias_v[pl.ds((f * _D + j) * _L, _L)]
            for k in range(8):
                val = plsc.load_gather(wide_v, [k * _L + lanes, slot[k] + j])
                tiles_v[j, pl.ds(k * _L, _L)] = val + bvec
            return c2
        lax.fori_loop(0, _D, pick, 0)
        def flush(tr, c2):
            pltpu.sync_copy(tiles_v.at[pl.ds(tr * 8, 8), :],
                            out2.at[pl.ds(f * _D + tr * 8, 8),
                                    pl.ds(tc * 128, 128)])
            return c2
        lax.fori_loop(0, 4, flush, 0)
        return carry

    lax.fori_loop(0, _PER_W, pair, 0)


@jax.jit
def kernel(x, emb, bias):
    x_flat = x.T.reshape(_F * _B)
    emb_p = emb.reshape(_RP, 128)
    bias_bc = jnp.repeat(bias.reshape(_F * _D), _L)

    mesh = plsc.VectorSubcoreMesh(core_axis_name="c", subcore_axis_name="s")
    run = functools.partial(
        pl.kernel,
        out_type=jax.ShapeDtypeStruct((_F, _D, _B), jnp.float32),
        mesh=mesh,
        compiler_params=pltpu.CompilerParams(use_tc_tiling_on_sc=True,
                                             needs_layout_passes=False),
        scratch_types=[
            pltpu.VMEM((128,), jnp.float32),
            pltpu.VMEM((1, 128), jnp.int32),
            pltpu.VMEM((128, 128), jnp.float32),
            pltpu.VMEM((_D, 128), jnp.float32),
            pltpu.VMEM((_F * _D * _L,), jnp.float32),
            pltpu.SemaphoreType.DMA,
        ],
    )(_body)
    out_t = run(x_flat, bias_bc, emb_p)
    return jnp.transpose(out_t, (2, 0, 1))
